# Initial kernel scaffold; baseline (speedup 1.0000x reference)
import jax
import jax.numpy as jnp
from jax.experimental import pallas as pl


def kernel(hidden_states, gate_weight, gate_up_weights, down_weights):
    raise NotImplementedError("write your pallas kernel here")


# speedup vs baseline: 1.3178x; 1.3178x over previous
import functools

import jax
import jax.numpy as jnp
from jax import lax
from jax.experimental import pallas as pl
from jax.experimental.pallas import tpu as pltpu
from jax.experimental.pallas import tpu_sc as plsc

D = 2048
I = 1408
E = 8
T = 4096
A = 2 * T
BM = 256
CAP = A + E * BM
NB = CAP // BM
NW = 32
CH = 16


def _router_body(x_ref, gw_ref, ids_ref, w_ref):
    l = jnp.dot(x_ref[...], gw_ref[...], preferred_element_type=jnp.float32)
    idx8 = lax.broadcasted_iota(jnp.int32, l.shape, 1)
    m0 = jnp.max(l, axis=-1, keepdims=True)
    e0 = jnp.min(jnp.where(l == m0, idx8, E), axis=-1, keepdims=True)
    lm = jnp.where(idx8 == e0, -jnp.inf, l)
    m1 = jnp.max(lm, axis=-1, keepdims=True)
    e1 = jnp.min(jnp.where(lm == m1, idx8, E), axis=-1, keepdims=True)
    p1 = jnp.exp(m1 - m0)
    w0 = 1.0 / (1.0 + p1)
    ids_ref[...] = jnp.concatenate([e0, e1], axis=1)
    w_ref[...] = jnp.concatenate([w0, 1.0 - w0], axis=1)


def _router(x, gw_t):
    bt = 1024
    return pl.pallas_call(
        _router_body,
        grid=(T // bt,),
        in_specs=[
            pl.BlockSpec((bt, D), lambda i: (i, 0)),
            pl.BlockSpec((D, E), lambda i: (0, 0)),
        ],
        out_specs=[
            pl.BlockSpec((bt, 2), lambda i: (i, 0)),
            pl.BlockSpec((bt, 2), lambda i: (i, 0)),
        ],
        out_shape=[
            jax.ShapeDtypeStruct((T, 2), jnp.int32),
            jax.ShapeDtypeStruct((T, 2), jnp.float32),
        ],
    )(x, gw_t)


def _dispatch_body(ef_ref, dest_ref, meta_ref):
    ef = ef_ref[...]
    r128 = lax.broadcasted_iota(jnp.int32, (128, 128), 0)
    c128 = lax.broadcasted_iota(jnp.int32, (128, 128), 1)
    upper = (r128 <= c128).astype(jnp.float32)
    r64 = lax.broadcasted_iota(jnp.int32, (64, 64), 0)
    c64 = lax.broadcasted_iota(jnp.int32, (64, 64), 1)
    lower = (c64 < r64).astype(jnp.float32)

    masks, incls, cnts = [], [], []
    for e in range(E):
        a = (ef == e).astype(jnp.float32)
        incl_row = jnp.dot(a, upper, preferred_element_type=jnp.float32)
        row_sum = jnp.sum(a, axis=1, keepdims=True)
        prev_rows = jnp.dot(lower, row_sum, preferred_element_type=jnp.float32)
        masks.append(a)
        incls.append(incl_row + prev_rows)
        cnts.append(jnp.sum(a).astype(jnp.int32))

    off = jnp.int32(0)
    ends = []
    dest = jnp.zeros((64, 128), jnp.int32)
    for e in range(E):
        dest = dest + masks[e].astype(jnp.int32) * (
            off + incls[e].astype(jnp.int32) - 1)
        off = off + ((cnts[e] + BM - 1) // BM) * BM
        ends.append(off)
    nreal = off // BM

    bstart = lax.broadcasted_iota(jnp.int32, (8, 128), 1) * BM
    be = jnp.zeros((8, 128), jnp.int32)
    for e in range(E):
        be = be + (bstart >= ends[e]).astype(jnp.int32)
    be = jnp.minimum(be, E - 1)
    rowi = lax.broadcasted_iota(jnp.int32, (8, 128), 0)
    dest_ref[...] = dest
    meta_ref[...] = jnp.where(rowi == 1, nreal, be)


def _dispatch(efr):
    return pl.pallas_call(
        _dispatch_body,
        out_shape=[
            jax.ShapeDtypeStruct((64, 128), jnp.int32),
            jax.ShapeDtypeStruct((8, 128), jnp.int32),
        ],
    )(efr)


def _sc_scatter(x, tok, dest):
    mesh = plsc.VectorSubcoreMesh(
        core_axis_name="c", subcore_axis_name="s", num_cores=2,
        num_subcores=16)

    @functools.partial(
        pl.kernel,
        out_type=jax.ShapeDtypeStruct((CAP, D), jnp.float32),
        mesh=mesh,
        scratch_types=[
            pltpu.VMEM((CH,), jnp.int32),
            pltpu.VMEM((CH,), jnp.int32),
            pltpu.VMEM((CH, D), jnp.float32),
            pltpu.SemaphoreType.DMA,
        ],
    )
    def k(x_hbm, tok_hbm, dest_hbm, xs_hbm, idx_t, idx_d, rows, sem):
        wid = lax.axis_index("s") * 2 + lax.axis_index("c")
        for c in range(A // (NW * CH)):
            base = wid * (A // NW) + c * CH
            pltpu.sync_copy(tok_hbm.at[pl.ds(base, CH)], idx_t)
            pltpu.sync_copy(dest_hbm.at[pl.ds(base, CH)], idx_d)
            pltpu.async_copy(x_hbm.at[idx_t], rows, sem).wait()
            pltpu.async_copy(rows, xs_hbm.at[idx_d], sem).wait()

    return k(x, tok, dest)


def _moe_body(pref_ref, xs_ref, wg_ref, wu_ref, wd_ref, ys_ref):
    m = pl.program_id(0)

    @pl.when(m < pref_ref[NB])
    def _():
        xb = xs_ref[...].astype(jnp.bfloat16)
        g = jnp.dot(xb, wg_ref[0], preferred_element_type=jnp.float32)
        u = jnp.dot(xb, wu_ref[0], preferred_element_type=jnp.float32)
        act = (g / (1.0 + jnp.exp(-g))) * u
        ys_ref[...] = jnp.dot(act.astype(jnp.bfloat16), wd_ref[0],
                              preferred_element_type=jnp.float32)


def _moe(pref, xs, wgb, wub, wdb):
    grid_spec = pltpu.PrefetchScalarGridSpec(
        num_scalar_prefetch=1,
        grid=(NB,),
        in_specs=[
            pl.BlockSpec((BM, D), lambda m, p: (m, 0)),
            pl.BlockSpec((1, D, I), lambda m, p: (p[m], 0, 0)),
            pl.BlockSpec((1, D, I), lambda m, p: (p[m], 0, 0)),
            pl.BlockSpec((1, I, D), lambda m, p: (p[m], 0, 0)),
        ],
        out_specs=pl.BlockSpec((BM, D), lambda m, p: (m, 0)),
    )
    return pl.pallas_call(
        _moe_body,
        grid_spec=grid_spec,
        out_shape=jax.ShapeDtypeStruct((CAP, D), jnp.float32),
    )(pref, xs, wgb, wub, wdb)


def _sc_gather(ys, dest):
    mesh = plsc.VectorSubcoreMesh(
        core_axis_name="c", subcore_axis_name="s", num_cores=2,
        num_subcores=16)

    @functools.partial(
        pl.kernel,
        out_type=jax.ShapeDtypeStruct((A, D), jnp.float32),
        mesh=mesh,
        scratch_types=[
            pltpu.VMEM((CH,), jnp.int32),
            pltpu.VMEM((CH, D), jnp.float32),
            pltpu.SemaphoreType.DMA,
        ],
    )
    def k(ys_hbm, dest_hbm, yp_hbm, idx_d, rows, sem):
        wid = lax.axis_index("s") * 2 + lax.axis_index("c")
        for c in range(A // (NW * CH)):
            base = wid * (A // NW) + c * CH
            pltpu.sync_copy(dest_hbm.at[pl.ds(base, CH)], idx_d)
            pltpu.async_copy(ys_hbm.at[idx_d], rows, sem).wait()
            pltpu.sync_copy(rows, yp_hbm.at[pl.ds(base, CH)])

    return k(ys, dest)


def _combine_body(yp_ref, w_ref, o_ref):
    w = w_ref[...]
    o_ref[...] = (yp_ref[:, 0, :] * w[:, 0:1] +
                  yp_ref[:, 1, :] * w[:, 1:2])


def _combine(yp3, w):
    bt = 512
    return pl.pallas_call(
        _combine_body,
        grid=(T // bt,),
        in_specs=[
            pl.BlockSpec((bt, 2, D), lambda i: (i, 0, 0)),
            pl.BlockSpec((bt, 2), lambda i: (i, 0)),
        ],
        out_specs=pl.BlockSpec((bt, D), lambda i: (i, 0)),
        out_shape=jax.ShapeDtypeStruct((T, D), jnp.float32),
    )(yp3, w)


def kernel(hidden_states, gate_weight, gate_up_weights, down_weights):
    x = hidden_states.reshape(-1, D)
    gw_t = gate_weight.T
    wgb = gate_up_weights[:, :, :I].astype(jnp.bfloat16)
    wub = gate_up_weights[:, :, I:].astype(jnp.bfloat16)
    wdb = down_weights.astype(jnp.bfloat16)

    ids, w = _router(x, gw_t)
    dest2, meta = _dispatch(ids.reshape(64, 128))
    dest = dest2.reshape(A)
    pref = jnp.concatenate([meta[0, :NB], meta[1, :1]]).astype(jnp.int32)
    tok = jnp.arange(A, dtype=jnp.int32) // 2

    xs = _sc_scatter(x, tok, dest)
    ys = _moe(pref, xs, wgb, wub, wdb)
    yp = _sc_gather(ys, dest)
    out = _combine(yp.reshape(T, 2, D), w)
    return out.reshape(hidden_states.shape)
